# Initial kernel scaffold; baseline (speedup 1.0000x reference)
import jax
import jax.numpy as jnp
from jax.experimental import pallas as pl


def kernel(prob_E):
    raise NotImplementedError("write your pallas kernel here")


# speedup vs baseline: 3.0170x; 3.0170x over previous
import numpy as np
import jax
import jax.numpy as jnp
from jax import lax
from jax.experimental import pallas as pl

N = 4096
BLK = 256
GRID = N // BLK

_K0 = np.uint32(0)
_K1 = np.uint32(42)
_K2 = np.uint32(0 ^ 42 ^ 0x1BD11BDA)
_ROT_A = (13, 15, 26, 6)
_ROT_B = (17, 29, 16, 24)


def _rotl(x, d):
    return lax.shift_left(x, np.uint32(d)) | lax.shift_right_logical(
        x, np.uint32(32 - d)
    )


def _rounds(x0, x1, rots):
    for d in rots:
        x0 = x0 + x1
        x1 = _rotl(x1, d)
        x1 = x1 ^ x0
    return x0, x1


def _threefry_bits(lo):
    x0 = jnp.zeros_like(lo)
    x1 = lo + _K1
    x0, x1 = _rounds(x0, x1, _ROT_A)
    x0 = x0 + _K1
    x1 = x1 + np.uint32((int(_K2) + 1) & 0xFFFFFFFF)
    x0, x1 = _rounds(x0, x1, _ROT_B)
    x0 = x0 + _K2
    x1 = x1 + np.uint32((int(_K0) + 2) & 0xFFFFFFFF)
    x0, x1 = _rounds(x0, x1, _ROT_A)
    x0 = x0 + _K0
    x1 = x1 + np.uint32((int(_K1) + 3) & 0xFFFFFFFF)
    x0, x1 = _rounds(x0, x1, _ROT_B)
    x0 = x0 + _K1
    x1 = x1 + np.uint32((int(_K2) + 4) & 0xFFFFFFFF)
    x0, x1 = _rounds(x0, x1, _ROT_A)
    x0 = x0 + _K2
    x1 = x1 + np.uint32((int(_K0) + 5) & 0xFFFFFFFF)
    return x0 ^ x1


_TINY = np.float32(np.finfo(np.float32).tiny)
_ONE_MINUS_TINY = np.float32(np.float32(1.0) - _TINY)


def _gumbel(bits):
    fb = lax.bitcast_convert_type(
        (bits >> np.uint32(9)) | np.uint32(0x3F800000), jnp.float32
    ) - np.float32(1.0)
    u = jnp.maximum(_TINY, fb * _ONE_MINUS_TINY + _TINY)
    return -jnp.log(-jnp.log(u))


def _sample_block(p_int, src_r0, src_c0, blk):
    rr = lax.broadcasted_iota(jnp.int32, (blk, blk * 2), 0)
    cj = lax.broadcasted_iota(jnp.int32, (blk, blk * 2), 1)
    k = (((src_r0 + rr) * N + src_c0) * 2 + cj).astype(jnp.uint32)
    y = jnp.log(p_int + np.float32(1e-30)) + _gumbel(_threefry_bits(k))
    z = jnp.concatenate([y[:, 1:], y[:, :1]], axis=1)
    w = (z > y).astype(jnp.bfloat16)
    a_io = lax.broadcasted_iota(jnp.int32, (blk * 2, blk), 0)
    j_io = lax.broadcasted_iota(jnp.int32, (blk * 2, blk), 1)
    sel = (a_io == j_io * 2).astype(jnp.bfloat16)
    sf = lax.dot_general(
        w, sel, (((1,), (0,)), ((), ())), preferred_element_type=jnp.float32
    )
    return (sf > np.float32(0.5)).astype(jnp.int32)


def _kernel(p_ref, out_ref):
    bi = pl.program_id(0)
    bj = pl.program_id(1)
    m = jnp.minimum(bi, bj)
    mm = jnp.maximum(bi, bj)
    s = _sample_block(p_ref[...], m * BLK, mm * BLK, BLK)
    rr = lax.broadcasted_iota(jnp.int32, (BLK, BLK), 0)
    cc = lax.broadcasted_iota(jnp.int32, (BLK, BLK), 1)
    upper = (bi * BLK + rr) <= (bj * BLK + cc)
    out_ref[...] = jnp.where(upper, s, s.T)


def kernel(prob_E):
    pr = prob_E.reshape(N, 2 * N)
    return pl.pallas_call(
        _kernel,
        grid=(GRID, GRID),
        in_specs=[
            pl.BlockSpec(
                (BLK, 2 * BLK),
                lambda i, j: (jnp.minimum(i, j), jnp.maximum(i, j)),
            )
        ],
        out_specs=pl.BlockSpec((BLK, BLK), lambda i, j: (i, j)),
        out_shape=jax.ShapeDtypeStruct((N, N), jnp.int32),
    )(pr)


# speedup vs baseline: 3.5568x; 1.1789x over previous
import numpy as np
import jax
import jax.numpy as jnp
from jax import lax
from jax.experimental import pallas as pl

N = 4096
BLK = 256
GRID = N // BLK

_K0 = np.uint32(0)
_K1 = np.uint32(42)
_K2 = np.uint32(0 ^ 42 ^ 0x1BD11BDA)
_ROT_A = (13, 15, 26, 6)
_ROT_B = (17, 29, 16, 24)


def _rotl(x, d):
    return lax.shift_left(x, np.uint32(d)) | lax.shift_right_logical(
        x, np.uint32(32 - d)
    )


def _rounds(x0, x1, rots):
    for d in rots:
        x0 = x0 + x1
        x1 = _rotl(x1, d)
        x1 = x1 ^ x0
    return x0, x1


def _threefry_bits(lo):
    x0 = jnp.zeros_like(lo)
    x1 = lo + _K1
    x0, x1 = _rounds(x0, x1, _ROT_A)
    x0 = x0 + _K1
    x1 = x1 + np.uint32((int(_K2) + 1) & 0xFFFFFFFF)
    x0, x1 = _rounds(x0, x1, _ROT_B)
    x0 = x0 + _K2
    x1 = x1 + np.uint32((int(_K0) + 2) & 0xFFFFFFFF)
    x0, x1 = _rounds(x0, x1, _ROT_A)
    x0 = x0 + _K0
    x1 = x1 + np.uint32((int(_K1) + 3) & 0xFFFFFFFF)
    x0, x1 = _rounds(x0, x1, _ROT_B)
    x0 = x0 + _K1
    x1 = x1 + np.uint32((int(_K2) + 4) & 0xFFFFFFFF)
    x0, x1 = _rounds(x0, x1, _ROT_A)
    x0 = x0 + _K2
    x1 = x1 + np.uint32((int(_K0) + 5) & 0xFFFFFFFF)
    return x0 ^ x1


_TINY = np.float32(np.finfo(np.float32).tiny)
_ONE_MINUS_TINY = np.float32(np.float32(1.0) - _TINY)


def _gumbel(bits):
    fb = lax.bitcast_convert_type(
        (bits >> np.uint32(9)) | np.uint32(0x3F800000), jnp.float32
    ) - np.float32(1.0)
    u = jnp.maximum(_TINY, fb * _ONE_MINUS_TINY + _TINY)
    return -jnp.log(-jnp.log(u))


def _sample_block(p_int, src_r0, src_c0, blk):
    rr = lax.broadcasted_iota(jnp.int32, (blk, blk * 2), 0)
    cj = lax.broadcasted_iota(jnp.int32, (blk, blk * 2), 1)
    k = (((src_r0 + rr) * N + src_c0) * 2 + cj).astype(jnp.uint32)
    y = jnp.log(p_int + np.float32(1e-30)) + _gumbel(_threefry_bits(k))
    z = jnp.concatenate([y[:, 1:], y[:, :1]], axis=1)
    w = (z > y).astype(jnp.bfloat16)
    a_io = lax.broadcasted_iota(jnp.int32, (blk * 2, blk), 0)
    j_io = lax.broadcasted_iota(jnp.int32, (blk * 2, blk), 1)
    sel = (a_io == j_io * 2).astype(jnp.bfloat16)
    sf = lax.dot_general(
        w, sel, (((1,), (0,)), ((), ())), preferred_element_type=jnp.float32
    )
    return (sf > np.float32(0.5)).astype(jnp.int32)


def _pass1(bi_ref, bj_ref, p_ref, ps_ref):
    u = pl.program_id(0)
    bi = bi_ref[u]
    bj = bj_ref[u]
    s = _sample_block(p_ref[...], bi * BLK, bj * BLK, BLK)
    ps_ref[0] = s.astype(jnp.int8)


def _pass2(w_ref, ps_ref, out_ref):
    del w_ref
    bi = pl.program_id(0)
    bj = pl.program_id(1)
    s = ps_ref[0].astype(jnp.int32)
    rr = lax.broadcasted_iota(jnp.int32, (BLK, BLK), 0)
    cc = lax.broadcasted_iota(jnp.int32, (BLK, BLK), 1)
    upper = (bi * BLK + rr) <= (bj * BLK + cc)
    out_ref[...] = jnp.where(upper, s, s.T)


_IU, _JU = np.triu_indices(GRID)
_NPAIRS = len(_IU)
_WTAB = np.zeros((GRID, GRID), np.int32)
_WTAB[_IU, _JU] = np.arange(_NPAIRS, dtype=np.int32)
_WTAB = np.maximum(_WTAB, _WTAB.T)

from jax.experimental.pallas import tpu as pltpu


def kernel(prob_E):
    pr = prob_E.reshape(N, 2 * N)
    ps = pl.pallas_call(
        _pass1,
        grid_spec=pltpu.PrefetchScalarGridSpec(
            num_scalar_prefetch=2,
            grid=(_NPAIRS,),
            in_specs=[
                pl.BlockSpec(
                    (BLK, 2 * BLK), lambda u, bi, bj: (bi[u], bj[u])
                )
            ],
            out_specs=pl.BlockSpec(
                (1, BLK, BLK), lambda u, bi, bj: (u, 0, 0)
            ),
        ),
        out_shape=jax.ShapeDtypeStruct((_NPAIRS, BLK, BLK), jnp.int8),
    )(jnp.asarray(_IU, jnp.int32), jnp.asarray(_JU, jnp.int32), pr)
    return pl.pallas_call(
        _pass2,
        grid_spec=pltpu.PrefetchScalarGridSpec(
            num_scalar_prefetch=1,
            grid=(GRID, GRID),
            in_specs=[
                pl.BlockSpec((1, BLK, BLK), lambda i, j, w: (w[i, j], 0, 0))
            ],
            out_specs=pl.BlockSpec((BLK, BLK), lambda i, j, w: (i, j)),
        ),
        out_shape=jax.ShapeDtypeStruct((N, N), jnp.int32),
    )(jnp.asarray(_WTAB), ps)


# speedup vs baseline: 6.1116x; 1.7183x over previous
import numpy as np
import jax
import jax.numpy as jnp
from jax import lax
from jax.experimental import pallas as pl
from jax.experimental.pallas import tpu as pltpu

N = 4096
BLK = 512
GRID = N // BLK

_K0 = np.uint32(0)
_K1 = np.uint32(42)
_K2 = np.uint32(0 ^ 42 ^ 0x1BD11BDA)
_ROT_A = (13, 15, 26, 6)
_ROT_B = (17, 29, 16, 24)


def _rotl(x, d):
    return lax.shift_left(x, np.uint32(d)) | lax.shift_right_logical(
        x, np.uint32(32 - d)
    )


def _rounds(x0, x1, rots):
    for d in rots:
        x0 = x0 + x1
        x1 = _rotl(x1, d)
        x1 = x1 ^ x0
    return x0, x1


def _threefry_bits(lo):
    x0 = jnp.zeros_like(lo)
    x1 = lo + _K1
    x0, x1 = _rounds(x0, x1, _ROT_A)
    x0 = x0 + _K1
    x1 = x1 + np.uint32((int(_K2) + 1) & 0xFFFFFFFF)
    x0, x1 = _rounds(x0, x1, _ROT_B)
    x0 = x0 + _K2
    x1 = x1 + np.uint32((int(_K0) + 2) & 0xFFFFFFFF)
    x0, x1 = _rounds(x0, x1, _ROT_A)
    x0 = x0 + _K0
    x1 = x1 + np.uint32((int(_K1) + 3) & 0xFFFFFFFF)
    x0, x1 = _rounds(x0, x1, _ROT_B)
    x0 = x0 + _K1
    x1 = x1 + np.uint32((int(_K2) + 4) & 0xFFFFFFFF)
    x0, x1 = _rounds(x0, x1, _ROT_A)
    x0 = x0 + _K2
    x1 = x1 + np.uint32((int(_K0) + 5) & 0xFFFFFFFF)
    return x0 ^ x1


_TINY = np.float32(np.finfo(np.float32).tiny)
_ONE_MINUS_TINY = np.float32(np.float32(1.0) - _TINY)


def _gumbel(bits):
    fb = lax.bitcast_convert_type(
        (bits >> np.uint32(9)) | np.uint32(0x3F800000), jnp.float32
    ) - np.float32(1.0)
    u = jnp.maximum(_TINY, fb * _ONE_MINUS_TINY + _TINY)
    return -jnp.log(-jnp.log(u))


def _y_slice(p, r0, q0, c):
    rr = lax.broadcasted_iota(jnp.int32, (BLK, 128), 0)
    ll = lax.broadcasted_iota(jnp.int32, (BLK, 128), 1)
    k = (((r0 + rr) * N + q0 + ll) * 2 + c).astype(jnp.uint32)
    return jnp.log(p + np.float32(1e-30)) + _gumbel(_threefry_bits(k))


def _pass1(bi_ref, bj_ref, p_ref, ps_ref):
    u = pl.program_id(0)
    bi = bi_ref[u]
    bj = bj_ref[u]
    r0 = bi * BLK
    for a in range(BLK // 128):
        q0 = bj * BLK + a * 128
        y0 = _y_slice(p_ref[:, 2 * a, :], r0, q0, 0)
        y1 = _y_slice(p_ref[:, 2 * a + 1, :], r0, q0, 1)
        ps_ref[0, :, a * 128 : (a + 1) * 128] = (y1 > y0).astype(jnp.int32)


def _pass2(w_ref, ps_ref, out_ref):
    del w_ref
    bi = pl.program_id(0)
    bj = pl.program_id(1)
    s = ps_ref[0]
    rr = lax.broadcasted_iota(jnp.int32, (BLK, BLK), 0)
    cc = lax.broadcasted_iota(jnp.int32, (BLK, BLK), 1)
    upper = (bi * BLK + rr) <= (bj * BLK + cc)
    out_ref[...] = jnp.where(upper, s, s.T)


_IU, _JU = np.triu_indices(GRID)
_NPAIRS = len(_IU)
_WTAB = np.zeros((GRID, GRID), np.int32)
_WTAB[_IU, _JU] = np.arange(_NPAIRS, dtype=np.int32)
_WTAB = np.maximum(_WTAB, _WTAB.T)


def kernel(prob_E):
    pr = prob_E.reshape(N, N // 128, 128, 2).transpose(0, 1, 3, 2)
    pr = pr.reshape(N, (N // 128) * 2, 128)
    ps = pl.pallas_call(
        _pass1,
        grid_spec=pltpu.PrefetchScalarGridSpec(
            num_scalar_prefetch=2,
            grid=(_NPAIRS,),
            in_specs=[
                pl.BlockSpec(
                    (BLK, (BLK // 128) * 2, 128),
                    lambda u, bi, bj: (bi[u], bj[u], 0),
                )
            ],
            out_specs=pl.BlockSpec(
                (1, BLK, BLK), lambda u, bi, bj: (u, 0, 0)
            ),
        ),
        out_shape=jax.ShapeDtypeStruct((_NPAIRS, BLK, BLK), jnp.int32),
    )(jnp.asarray(_IU, jnp.int32), jnp.asarray(_JU, jnp.int32), pr)
    return pl.pallas_call(
        _pass2,
        grid_spec=pltpu.PrefetchScalarGridSpec(
            num_scalar_prefetch=1,
            grid=(GRID, GRID),
            in_specs=[
                pl.BlockSpec((1, BLK, BLK), lambda i, j, w: (w[i, j], 0, 0))
            ],
            out_specs=pl.BlockSpec((BLK, BLK), lambda i, j, w: (i, j)),
        ),
        out_shape=jax.ShapeDtypeStruct((N, N), jnp.int32),
    )(jnp.asarray(_WTAB), ps)


# speedup vs baseline: 6.1286x; 1.0028x over previous
import numpy as np
import jax
import jax.numpy as jnp
from jax import lax
from jax.experimental import pallas as pl
from jax.experimental.pallas import tpu as pltpu

N = 4096
BLK = 512
GRID = N // BLK

_K0 = np.uint32(0)
_K1 = np.uint32(42)
_K2 = np.uint32(0 ^ 42 ^ 0x1BD11BDA)
_ROT_A = (13, 15, 26, 6)
_ROT_B = (17, 29, 16, 24)


def _rotl(x, d):
    return lax.shift_left(x, np.uint32(d)) | lax.shift_right_logical(
        x, np.uint32(32 - d)
    )


def _rounds(x0, x1, rots):
    for d in rots:
        x0 = x0 + x1
        x1 = _rotl(x1, d)
        x1 = x1 ^ x0
    return x0, x1


def _threefry_bits(lo):
    x0 = jnp.zeros_like(lo)
    x1 = lo + _K1
    x0, x1 = _rounds(x0, x1, _ROT_A)
    x0 = x0 + _K1
    x1 = x1 + np.uint32((int(_K2) + 1) & 0xFFFFFFFF)
    x0, x1 = _rounds(x0, x1, _ROT_B)
    x0 = x0 + _K2
    x1 = x1 + np.uint32((int(_K0) + 2) & 0xFFFFFFFF)
    x0, x1 = _rounds(x0, x1, _ROT_A)
    x0 = x0 + _K0
    x1 = x1 + np.uint32((int(_K1) + 3) & 0xFFFFFFFF)
    x0, x1 = _rounds(x0, x1, _ROT_B)
    x0 = x0 + _K1
    x1 = x1 + np.uint32((int(_K2) + 4) & 0xFFFFFFFF)
    x0, x1 = _rounds(x0, x1, _ROT_A)
    x0 = x0 + _K2
    x1 = x1 + np.uint32((int(_K0) + 5) & 0xFFFFFFFF)
    return x0 ^ x1


_TINY = np.float32(np.finfo(np.float32).tiny)
_ONE_MINUS_TINY = np.float32(np.float32(1.0) - _TINY)


def _gumbel(bits):
    fb = lax.bitcast_convert_type(
        (bits >> np.uint32(9)) | np.uint32(0x3F800000), jnp.float32
    ) - np.float32(1.0)
    u = jnp.maximum(_TINY, fb * _ONE_MINUS_TINY + _TINY)
    return -jnp.log(-jnp.log(u))


def _y_slice(p, r0, q0, c):
    rr = lax.broadcasted_iota(jnp.int32, (BLK, 128), 0)
    ll = lax.broadcasted_iota(jnp.int32, (BLK, 128), 1)
    k = (((r0 + rr) * N + q0 + ll) * 2 + c).astype(jnp.uint32)
    return jnp.log(p + np.float32(1e-30)) + _gumbel(_threefry_bits(k))


def _fused(bi_ref, bj_ref, typ_ref, p_ref, out_ref, s_ref):
    u = pl.program_id(0)
    bi = bi_ref[u]
    bj = bj_ref[u]
    typ = typ_ref[u]

    @pl.when(typ < 2)
    def _compute():
        r0 = bi * BLK
        for a in range(BLK // 128):
            q0 = bj * BLK + a * 128
            y0 = _y_slice(p_ref[:, 2 * a, :], r0, q0, 0)
            y1 = _y_slice(p_ref[:, 2 * a + 1, :], r0, q0, 1)
            s_ref[:, a * 128 : (a + 1) * 128] = (y1 > y0).astype(jnp.int32)

    s = s_ref[...]

    @pl.when(typ == 0)
    def _emit_upper():
        out_ref[...] = s

    @pl.when(typ == 1)
    def _emit_diag():
        rr = lax.broadcasted_iota(jnp.int32, (BLK, BLK), 0)
        cc = lax.broadcasted_iota(jnp.int32, (BLK, BLK), 1)
        out_ref[...] = jnp.where(rr <= cc, s, s.T)

    @pl.when(typ == 2)
    def _emit_mirror():
        out_ref[...] = s.T


_STEPS = []
for _i in range(GRID):
    _STEPS.append((_i, _i, 1))
for _i in range(GRID):
    for _j in range(_i + 1, GRID):
        _STEPS.append((_i, _j, 0))
        _STEPS.append((_j, _i, 2))
_BI = np.array([s[0] for s in _STEPS], np.int32)
_BJ = np.array([s[1] for s in _STEPS], np.int32)
_TY = np.array([s[2] for s in _STEPS], np.int32)


def kernel(prob_E):
    pr = prob_E.reshape(N, N // 128, 128, 2).transpose(0, 1, 3, 2)
    pr = pr.reshape(N, (N // 128) * 2, 128)
    return pl.pallas_call(
        _fused,
        grid_spec=pltpu.PrefetchScalarGridSpec(
            num_scalar_prefetch=3,
            grid=(len(_STEPS),),
            in_specs=[
                pl.BlockSpec(
                    (BLK, (BLK // 128) * 2, 128),
                    lambda u, bi, bj, ty: (
                        jnp.minimum(bi[u], bj[u]),
                        jnp.maximum(bi[u], bj[u]),
                        0,
                    ),
                )
            ],
            out_specs=pl.BlockSpec(
                (BLK, BLK), lambda u, bi, bj, ty: (bi[u], bj[u])
            ),
            scratch_shapes=[pltpu.VMEM((BLK, BLK), jnp.int32)],
        ),
        out_shape=jax.ShapeDtypeStruct((N, N), jnp.int32),
    )(
        jnp.asarray(_BI),
        jnp.asarray(_BJ),
        jnp.asarray(_TY),
        pr,
    )


# speedup vs baseline: 6.5838x; 1.0743x over previous
import numpy as np
import jax
import jax.numpy as jnp
from jax import lax
from jax.experimental import pallas as pl
from jax.experimental.pallas import tpu as pltpu

N = 4096
BLK = 512
GRID = N // BLK

_K0 = np.uint32(0)
_K1 = np.uint32(42)
_K2 = np.uint32(0 ^ 42 ^ 0x1BD11BDA)
_ROT_A = (13, 15, 26, 6)
_ROT_B = (17, 29, 16, 24)


def _rotl(x, d):
    return lax.shift_left(x, np.uint32(d)) | lax.shift_right_logical(
        x, np.uint32(32 - d)
    )


def _rounds(x0, x1, rots):
    for d in rots:
        x0 = x0 + x1
        x1 = _rotl(x1, d)
        x1 = x1 ^ x0
    return x0, x1


def _threefry_bits(lo):
    x0 = jnp.zeros_like(lo)
    x1 = lo + _K1
    x0, x1 = _rounds(x0, x1, _ROT_A)
    x0 = x0 + _K1
    x1 = x1 + np.uint32((int(_K2) + 1) & 0xFFFFFFFF)
    x0, x1 = _rounds(x0, x1, _ROT_B)
    x0 = x0 + _K2
    x1 = x1 + np.uint32((int(_K0) + 2) & 0xFFFFFFFF)
    x0, x1 = _rounds(x0, x1, _ROT_A)
    x0 = x0 + _K0
    x1 = x1 + np.uint32((int(_K1) + 3) & 0xFFFFFFFF)
    x0, x1 = _rounds(x0, x1, _ROT_B)
    x0 = x0 + _K1
    x1 = x1 + np.uint32((int(_K2) + 4) & 0xFFFFFFFF)
    x0, x1 = _rounds(x0, x1, _ROT_A)
    x0 = x0 + _K2
    x1 = x1 + np.uint32((int(_K0) + 5) & 0xFFFFFFFF)
    return x0 ^ x1


_TINY = np.float32(np.finfo(np.float32).tiny)
_ONE_MINUS_TINY = np.float32(np.float32(1.0) - _TINY)


def _gumbel(bits):
    fb = lax.bitcast_convert_type(
        (bits >> np.uint32(9)) | np.uint32(0x3F800000), jnp.float32
    ) - np.float32(1.0)
    u = jnp.maximum(_TINY, fb * _ONE_MINUS_TINY + _TINY)
    return -jnp.log(-jnp.log(u))


def _y_slice(p, r0, q0, c, rows):
    rr = lax.broadcasted_iota(jnp.int32, (rows, 128), 0)
    ll = lax.broadcasted_iota(jnp.int32, (rows, 128), 1)
    k = (((r0 + rr) * N + q0 + ll) * 2 + c).astype(jnp.uint32)
    return jnp.log(p + np.float32(1e-30)) + _gumbel(_threefry_bits(k))


def _fused(bi_ref, bj_ref, typ_ref, p_ref, out_ref, s_ref):
    u = pl.program_id(0)
    bi = bi_ref[u]
    bj = bj_ref[u]
    typ = typ_ref[u]

    @pl.when(typ == 0)
    def _compute():
        r0 = bi * BLK
        for a in range(BLK // 128):
            q0 = bj * BLK + a * 128
            y0 = _y_slice(p_ref[:, 2 * a, :], r0, q0, 0, BLK)
            y1 = _y_slice(p_ref[:, 2 * a + 1, :], r0, q0, 1, BLK)
            s_ref[:, a * 128 : (a + 1) * 128] = (y1 > y0).astype(jnp.int32)

    @pl.when(typ == 1)
    def _compute_diag():
        r0 = bi * BLK
        for a in range(BLK // 128):
            q0 = bj * BLK + a * 128
            rows = (a + 1) * 128
            y0 = _y_slice(p_ref[:rows, 2 * a, :], r0, q0, 0, rows)
            y1 = _y_slice(p_ref[:rows, 2 * a + 1, :], r0, q0, 1, rows)
            s_ref[:rows, a * 128 : (a + 1) * 128] = (y1 > y0).astype(
                jnp.int32
            )

    s = s_ref[...]

    @pl.when(typ == 0)
    def _emit_upper():
        out_ref[...] = s

    @pl.when(typ == 1)
    def _emit_diag():
        rr = lax.broadcasted_iota(jnp.int32, (BLK, BLK), 0)
        cc = lax.broadcasted_iota(jnp.int32, (BLK, BLK), 1)
        out_ref[...] = jnp.where(rr <= cc, s, s.T)

    @pl.when(typ == 2)
    def _emit_mirror():
        out_ref[...] = s.T


_STEPS = []
for _i in range(GRID):
    _STEPS.append((_i, _i, 1))
for _i in range(GRID):
    for _j in range(_i + 1, GRID):
        _STEPS.append((_i, _j, 0))
        _STEPS.append((_j, _i, 2))
_BI = np.array([s[0] for s in _STEPS], np.int32)
_BJ = np.array([s[1] for s in _STEPS], np.int32)
_TY = np.array([s[2] for s in _STEPS], np.int32)


def kernel(prob_E):
    pr = prob_E.reshape(N, N // 128, 128, 2).transpose(0, 1, 3, 2)
    pr = pr.reshape(N, (N // 128) * 2, 128)
    return pl.pallas_call(
        _fused,
        grid_spec=pltpu.PrefetchScalarGridSpec(
            num_scalar_prefetch=3,
            grid=(len(_STEPS),),
            in_specs=[
                pl.BlockSpec(
                    (BLK, (BLK // 128) * 2, 128),
                    lambda u, bi, bj, ty: (
                        jnp.minimum(bi[u], bj[u]),
                        jnp.maximum(bi[u], bj[u]),
                        0,
                    ),
                )
            ],
            out_specs=pl.BlockSpec(
                (BLK, BLK), lambda u, bi, bj, ty: (bi[u], bj[u])
            ),
            scratch_shapes=[pltpu.VMEM((BLK, BLK), jnp.int32)],
        ),
        out_shape=jax.ShapeDtypeStruct((N, N), jnp.int32),
    )(
        jnp.asarray(_BI),
        jnp.asarray(_BJ),
        jnp.asarray(_TY),
        pr,
    )


# speedup vs baseline: 7.2152x; 1.0959x over previous
import numpy as np
import jax
import jax.numpy as jnp
from jax import lax
from jax.experimental import pallas as pl
from jax.experimental.pallas import tpu as pltpu

N = 4096
BLK = 512
GRID = N // BLK

_K0 = np.uint32(0)
_K1 = np.uint32(42)
_K2 = np.uint32(0 ^ 42 ^ 0x1BD11BDA)
_ROT_A = (13, 15, 26, 6)
_ROT_B = (17, 29, 16, 24)


def _rotl(x, d):
    return lax.shift_left(x, np.uint32(d)) | lax.shift_right_logical(
        x, np.uint32(32 - d)
    )


def _rounds(x0, x1, rots):
    for d in rots:
        x0 = x0 + x1
        x1 = _rotl(x1, d)
        x1 = x1 ^ x0
    return x0, x1


def _threefry_bits(lo):
    x0 = jnp.zeros_like(lo)
    x1 = lo + _K1
    x0, x1 = _rounds(x0, x1, _ROT_A)
    x0 = x0 + _K1
    x1 = x1 + np.uint32((int(_K2) + 1) & 0xFFFFFFFF)
    x0, x1 = _rounds(x0, x1, _ROT_B)
    x0 = x0 + _K2
    x1 = x1 + np.uint32((int(_K0) + 2) & 0xFFFFFFFF)
    x0, x1 = _rounds(x0, x1, _ROT_A)
    x0 = x0 + _K0
    x1 = x1 + np.uint32((int(_K1) + 3) & 0xFFFFFFFF)
    x0, x1 = _rounds(x0, x1, _ROT_B)
    x0 = x0 + _K1
    x1 = x1 + np.uint32((int(_K2) + 4) & 0xFFFFFFFF)
    x0, x1 = _rounds(x0, x1, _ROT_A)
    x0 = x0 + _K2
    x1 = x1 + np.uint32((int(_K0) + 5) & 0xFFFFFFFF)
    return x0 ^ x1


_TINY = np.float32(np.finfo(np.float32).tiny)
_ONE_MINUS_TINY = np.float32(np.float32(1.0) - _TINY)


def _t_slice(r0, q0, c, rows):
    rr = lax.broadcasted_iota(jnp.int32, (rows, 128), 0)
    ll = lax.broadcasted_iota(jnp.int32, (rows, 128), 1)
    k = (((r0 + rr) * N + q0 + ll) * 2 + c).astype(jnp.uint32)
    bits = _threefry_bits(k)
    fb = lax.bitcast_convert_type(
        (bits >> np.uint32(9)) | np.uint32(0x3F800000), jnp.float32
    ) - np.float32(1.0)
    u = jnp.maximum(_TINY, fb * _ONE_MINUS_TINY + _TINY)
    return -jnp.log(u)


def _fused(bi_ref, bj_ref, typ_ref, p_ref, out_ref, s_ref):
    u = pl.program_id(0)
    bi = bi_ref[u]
    bj = bj_ref[u]
    typ = typ_ref[u]

    @pl.when(typ == 0)
    def _compute():
        r0 = bi * BLK
        eps = np.float32(1e-30)
        for a in range(BLK // 128):
            q0 = bj * BLK + a * 128
            t0 = _t_slice(r0, q0, 0, BLK)
            t1 = _t_slice(r0, q0, 1, BLK)
            x0 = (p_ref[:, 2 * a, :] + eps) * t1
            x1 = (p_ref[:, 2 * a + 1, :] + eps) * t0
            s_ref[:, a * 128 : (a + 1) * 128] = (x1 > x0).astype(jnp.int32)

    @pl.when(typ == 1)
    def _compute_diag():
        r0 = bi * BLK
        eps = np.float32(1e-30)
        for a in range(BLK // 128):
            q0 = bj * BLK + a * 128
            rows = (a + 1) * 128
            t0 = _t_slice(r0, q0, 0, rows)
            t1 = _t_slice(r0, q0, 1, rows)
            x0 = (p_ref[:rows, 2 * a, :] + eps) * t1
            x1 = (p_ref[:rows, 2 * a + 1, :] + eps) * t0
            s_ref[:rows, a * 128 : (a + 1) * 128] = (x1 > x0).astype(
                jnp.int32
            )

    s = s_ref[...]

    @pl.when(typ == 0)
    def _emit_upper():
        out_ref[...] = s

    @pl.when(typ == 1)
    def _emit_diag():
        rr = lax.broadcasted_iota(jnp.int32, (BLK, BLK), 0)
        cc = lax.broadcasted_iota(jnp.int32, (BLK, BLK), 1)
        out_ref[...] = jnp.where(rr <= cc, s, s.T)

    @pl.when(typ == 2)
    def _emit_mirror():
        out_ref[...] = s.T


_STEPS = []
for _i in range(GRID):
    _STEPS.append((_i, _i, 1))
for _i in range(GRID):
    for _j in range(_i + 1, GRID):
        _STEPS.append((_i, _j, 0))
        _STEPS.append((_j, _i, 2))
_BI = np.array([s[0] for s in _STEPS], np.int32)
_BJ = np.array([s[1] for s in _STEPS], np.int32)
_TY = np.array([s[2] for s in _STEPS], np.int32)


def kernel(prob_E):
    pr = prob_E.reshape(N, N // 128, 128, 2).transpose(0, 1, 3, 2)
    pr = pr.reshape(N, (N // 128) * 2, 128)
    return pl.pallas_call(
        _fused,
        grid_spec=pltpu.PrefetchScalarGridSpec(
            num_scalar_prefetch=3,
            grid=(len(_STEPS),),
            in_specs=[
                pl.BlockSpec(
                    (BLK, (BLK // 128) * 2, 128),
                    lambda u, bi, bj, ty: (
                        jnp.minimum(bi[u], bj[u]),
                        jnp.maximum(bi[u], bj[u]),
                        0,
                    ),
                )
            ],
            out_specs=pl.BlockSpec(
                (BLK, BLK), lambda u, bi, bj, ty: (bi[u], bj[u])
            ),
            scratch_shapes=[pltpu.VMEM((BLK, BLK), jnp.int32)],
        ),
        out_shape=jax.ShapeDtypeStruct((N, N), jnp.int32),
    )(
        jnp.asarray(_BI),
        jnp.asarray(_BJ),
        jnp.asarray(_TY),
        pr,
    )


# speedup vs baseline: 7.2586x; 1.0060x over previous
import numpy as np
import jax
import jax.numpy as jnp
from jax import lax
from jax.experimental import pallas as pl
from jax.experimental.pallas import tpu as pltpu

N = 4096
BLK = 512
GRID = N // BLK

_K0 = np.uint32(0)
_K1 = np.uint32(42)
_K2 = np.uint32(0 ^ 42 ^ 0x1BD11BDA)
_ROT_A = (13, 15, 26, 6)
_ROT_B = (17, 29, 16, 24)


def _rotl(x, d):
    return lax.shift_left(x, np.uint32(d)) | lax.shift_right_logical(
        x, np.uint32(32 - d)
    )


def _rounds(x0, x1, rots):
    for d in rots:
        x0 = x0 + x1
        x1 = _rotl(x1, d)
        x1 = x1 ^ x0
    return x0, x1


def _threefry_bits(lo):
    xi = lo + _K1
    x0 = xi
    x1 = _rotl(xi, 13) ^ xi
    x0, x1 = _rounds(x0, x1, _ROT_A[1:])
    x0 = x0 + _K1
    x1 = x1 + np.uint32((int(_K2) + 1) & 0xFFFFFFFF)
    x0, x1 = _rounds(x0, x1, _ROT_B)
    x0 = x0 + _K2
    x1 = x1 + np.uint32((int(_K0) + 2) & 0xFFFFFFFF)
    x0, x1 = _rounds(x0, x1, _ROT_A)
    x0 = x0 + _K0
    x1 = x1 + np.uint32((int(_K1) + 3) & 0xFFFFFFFF)
    x0, x1 = _rounds(x0, x1, _ROT_B)
    x0 = x0 + _K1
    x1 = x1 + np.uint32((int(_K2) + 4) & 0xFFFFFFFF)
    x0, x1 = _rounds(x0, x1, _ROT_A)
    x0 = x0 + _K2
    x1 = x1 + np.uint32((int(_K0) + 5) & 0xFFFFFFFF)
    return x0 ^ x1


_TINY = np.float32(np.finfo(np.float32).tiny)
_ONE_MINUS_TINY = np.float32(np.float32(1.0) - _TINY)


def _t_slice(kb, r0, q0, c):
    k = (kb + ((r0 * N + q0) * 2 + c)).astype(jnp.uint32)
    bits = _threefry_bits(k)
    fb = lax.bitcast_convert_type(
        (bits >> np.uint32(9)) | np.uint32(0x3F800000), jnp.float32
    ) - np.float32(1.0)
    u = jnp.maximum(_TINY, fb * _ONE_MINUS_TINY + _TINY)
    return -jnp.log(u)


def _fused(bi_ref, bj_ref, typ_ref, p_ref, out_ref, s_ref):
    u = pl.program_id(0)
    bi = bi_ref[u]
    bj = bj_ref[u]
    typ = typ_ref[u]

    @pl.when(typ == 0)
    def _compute():
        r0 = bi * BLK
        eps = np.float32(1e-30)
        kb = 2 * N * lax.broadcasted_iota(
            jnp.int32, (BLK, 128), 0
        ) + 2 * lax.broadcasted_iota(jnp.int32, (BLK, 128), 1)
        for a in range(BLK // 128):
            q0 = bj * BLK + a * 128
            t0 = _t_slice(kb, r0, q0, 0)
            t1 = _t_slice(kb, r0, q0, 1)
            x0 = (p_ref[:, 2 * a, :] + eps) * t1
            x1 = (p_ref[:, 2 * a + 1, :] + eps) * t0
            s_ref[:, a * 128 : (a + 1) * 128] = (x1 > x0).astype(jnp.int32)

    @pl.when(typ == 1)
    def _compute_diag():
        r0 = bi * BLK
        eps = np.float32(1e-30)
        kb_full = 2 * N * lax.broadcasted_iota(
            jnp.int32, (BLK, 128), 0
        ) + 2 * lax.broadcasted_iota(jnp.int32, (BLK, 128), 1)
        for a in range(BLK // 128):
            q0 = bj * BLK + a * 128
            rows = (a + 1) * 128
            t0 = _t_slice(kb_full[:rows], r0, q0, 0)
            t1 = _t_slice(kb_full[:rows], r0, q0, 1)
            x0 = (p_ref[:rows, 2 * a, :] + eps) * t1
            x1 = (p_ref[:rows, 2 * a + 1, :] + eps) * t0
            s_ref[:rows, a * 128 : (a + 1) * 128] = (x1 > x0).astype(
                jnp.int32
            )

    s = s_ref[...]

    @pl.when(typ == 0)
    def _emit_upper():
        out_ref[...] = s

    @pl.when(typ == 1)
    def _emit_diag():
        rr = lax.broadcasted_iota(jnp.int32, (BLK, BLK), 0)
        cc = lax.broadcasted_iota(jnp.int32, (BLK, BLK), 1)
        out_ref[...] = jnp.where(rr <= cc, s, s.T)

    @pl.when(typ == 2)
    def _emit_mirror():
        out_ref[...] = s.T


_STEPS = []
for _i in range(GRID):
    _STEPS.append((_i, _i, 1))
for _i in range(GRID):
    for _j in range(_i + 1, GRID):
        _STEPS.append((_i, _j, 0))
        _STEPS.append((_j, _i, 2))
_BI = np.array([s[0] for s in _STEPS], np.int32)
_BJ = np.array([s[1] for s in _STEPS], np.int32)
_TY = np.array([s[2] for s in _STEPS], np.int32)


def kernel(prob_E):
    pr = prob_E.reshape(N, N // 128, 128, 2).transpose(0, 1, 3, 2)
    pr = pr.reshape(N, (N // 128) * 2, 128)
    return pl.pallas_call(
        _fused,
        grid_spec=pltpu.PrefetchScalarGridSpec(
            num_scalar_prefetch=3,
            grid=(len(_STEPS),),
            in_specs=[
                pl.BlockSpec(
                    (BLK, (BLK // 128) * 2, 128),
                    lambda u, bi, bj, ty: (
                        jnp.minimum(bi[u], bj[u]),
                        jnp.maximum(bi[u], bj[u]),
                        0,
                    ),
                )
            ],
            out_specs=pl.BlockSpec(
                (BLK, BLK), lambda u, bi, bj, ty: (bi[u], bj[u])
            ),
            scratch_shapes=[pltpu.VMEM((BLK, BLK), jnp.int32)],
        ),
        out_shape=jax.ShapeDtypeStruct((N, N), jnp.int32),
    )(
        jnp.asarray(_BI),
        jnp.asarray(_BJ),
        jnp.asarray(_TY),
        pr,
    )
